# baseline (speedup 1.0000x reference)
import functools

import jax
import jax.numpy as jnp
from jax import lax
from jax.experimental import pallas as pl
from jax.experimental.pallas import tpu as pltpu
from jax.experimental.pallas import tpu_sc as plsc

_BN = 256
_AB = 256
_AP = 1024
_RW = 16


def _body(state_ref, w1_ref, b1_ref, w2_ref, b2_ref, w3_ref, b3_ref,
          v1_ref, vb1_ref, v2_ref, vb2_ref, v3_ref, vb3_ref, g_ref,
          probs_ref, value_ref, y_ref, lp_ref,
          sb, h1a, h2a, h1v, h2v, lg):
    B, S = state_ref.shape
    A = b3_ref.shape[1]
    H = h1a.shape[1]
    nb = H // _BN
    na = lg.shape[1] // _AB
    o2 = nb
    o3 = 2 * nb
    o4 = o3 + na

    i = pl.program_id(0)

    @pl.when(i == 0)
    def _cast_state():
        sb[...] = state_ref[...].astype(jnp.bfloat16)

    @pl.when(i < o2)
    def _p0():
        j = i
        xa = jnp.dot(sb[...], w1_ref[...], preferred_element_type=jnp.float32)
        h1a[:, pl.ds(j * _BN, _BN)] = jnp.tanh(xa + b1_ref[...]).astype(jnp.bfloat16)
        xv = jnp.dot(sb[...], v1_ref[...], preferred_element_type=jnp.float32)
        h1v[:, pl.ds(j * _BN, _BN)] = jnp.tanh(xv + vb1_ref[...]).astype(jnp.bfloat16)

    @pl.when((i >= o2) & (i < o3))
    def _p1():
        j = i - o2
        xa = jnp.dot(h1a[...], w2_ref[...], preferred_element_type=jnp.float32)
        h2a[:, pl.ds(j * _BN, _BN)] = jnp.tanh(xa + b2_ref[...]).astype(jnp.bfloat16)
        xv = jnp.dot(h1v[...], v2_ref[...], preferred_element_type=jnp.float32)
        h2v[:, pl.ds(j * _BN, _BN)] = jnp.tanh(xv + vb2_ref[...]).astype(jnp.bfloat16)

    @pl.when((i >= o3) & (i < o4))
    def _p2():
        j = i - o3
        lg[:, pl.ds(j * _AB, _AB)] = jnp.dot(
            h2a[...], w3_ref[...], preferred_element_type=jnp.float32)

    @pl.when(i == o4)
    def _fin():
        v3row = v3_ref[...].astype(jnp.bfloat16).astype(jnp.float32)
        hv = h2v[...].astype(jnp.float32)
        value_ref[...] = (jnp.sum(hv * v3row, axis=-1, keepdims=True)
                          + vb3_ref[...])
        logits = lg[:, :A] + b3_ref[...]
        m = jnp.max(logits, axis=-1, keepdims=True)
        e = jnp.exp(logits - m)
        p = e / jnp.sum(e, axis=-1, keepdims=True)
        probs_ref[...] = p
        lp = jnp.log(p + 1e-20)
        lp_ref[:, :A] = lp
        lp_ref[:, A:] = jnp.zeros((B, _AP - A), jnp.float32)
        y_ref[:, :A] = lp + g_ref[...]
        y_ref[:, A:] = jnp.full((B, _AP - A), -3.0e38, jnp.float32)


def _towers(state, W1, b1, W2, b2, W3, b3, V1, vb1, V2, vb2, V3, vb3, g):
    B, S = state.shape
    H = W1.shape[1]
    A = W3.shape[1]
    nb = H // _BN
    na = pl.cdiv(A, _AB)
    Ap = na * _AB
    o2, o3 = nb, 2 * nb
    o4 = o3 + na
    steps = o4 + 1

    in_specs = [
        pl.BlockSpec((B, S), lambda i: (0, 0)),
        pl.BlockSpec((S, _BN), lambda i: (0, jnp.clip(i, 0, nb - 1))),
        pl.BlockSpec((1, _BN), lambda i: (0, jnp.clip(i, 0, nb - 1))),
        pl.BlockSpec((H, _BN), lambda i: (0, jnp.clip(i - o2, 0, nb - 1))),
        pl.BlockSpec((1, _BN), lambda i: (0, jnp.clip(i - o2, 0, nb - 1))),
        pl.BlockSpec((S, _AB), lambda i: (0, jnp.clip(i - o3, 0, na - 1))),
        pl.BlockSpec((1, A), lambda i: (0, 0)),
        pl.BlockSpec((S, _BN), lambda i: (0, jnp.clip(i, 0, nb - 1))),
        pl.BlockSpec((1, _BN), lambda i: (0, jnp.clip(i, 0, nb - 1))),
        pl.BlockSpec((H, _BN), lambda i: (0, jnp.clip(i - o2, 0, nb - 1))),
        pl.BlockSpec((1, _BN), lambda i: (0, jnp.clip(i - o2, 0, nb - 1))),
        pl.BlockSpec((1, S), lambda i: (0, 0)),
        pl.BlockSpec((1, 1), lambda i: (0, 0)),
        pl.BlockSpec((B, A), lambda i: (0, 0)),
    ]
    out_specs = [
        pl.BlockSpec((B, A), lambda i: (0, 0)),
        pl.BlockSpec((B, 1), lambda i: (0, 0)),
        pl.BlockSpec((B, _AP), lambda i: (0, 0)),
        pl.BlockSpec((B, _AP), lambda i: (0, 0)),
    ]
    out_shape = [
        jax.ShapeDtypeStruct((B, A), jnp.float32),
        jax.ShapeDtypeStruct((B, 1), jnp.float32),
        jax.ShapeDtypeStruct((B, _AP), jnp.float32),
        jax.ShapeDtypeStruct((B, _AP), jnp.float32),
    ]
    scratch_shapes = [
        pltpu.VMEM((B, S), jnp.bfloat16),
        pltpu.VMEM((B, H), jnp.bfloat16),
        pltpu.VMEM((B, H), jnp.bfloat16),
        pltpu.VMEM((B, H), jnp.bfloat16),
        pltpu.VMEM((B, H), jnp.bfloat16),
        pltpu.VMEM((B, Ap), jnp.float32),
    ]

    return pl.pallas_call(
        _body,
        grid=(steps,),
        in_specs=in_specs,
        out_specs=out_specs,
        out_shape=out_shape,
        scratch_shapes=scratch_shapes,
    )(state, W1, b1.reshape(1, H), W2, b2.reshape(1, H),
      W3, b3.reshape(1, A), V1, vb1.reshape(1, H), V2, vb2.reshape(1, H),
      V3.reshape(1, S), vb3.reshape(1, 1), g)


def _sc_sample(y, lp, B, Ap):
    nworkers = B // _RW
    mesh = plsc.VectorSubcoreMesh(core_axis_name="c", subcore_axis_name="s")
    info = plsc.get_sparse_core_info()
    NC = info.num_cores

    @functools.partial(
        pl.kernel,
        out_type=[
            jax.ShapeDtypeStruct((B,), jnp.int32),
            jax.ShapeDtypeStruct((B,), jnp.float32),
        ],
        mesh=mesh,
        scratch_types=[
            pltpu.VMEM((_RW * Ap,), jnp.float32),
            pltpu.VMEM((_RW * Ap,), jnp.float32),
            pltpu.VMEM((_RW,), jnp.int32),
            pltpu.VMEM((_RW,), jnp.float32),
        ],
    )
    def samp(y_hbm, lp_hbm, act_hbm, alp_hbm, yv, lpv, actv, alpv):
        wid = lax.axis_index("s") * NC + lax.axis_index("c")

        @pl.when(wid < nworkers)
        def _work():
            base = wid * _RW
            pltpu.sync_copy(y_hbm.at[pl.ds(base * Ap, _RW * Ap)], yv)
            pltpu.sync_copy(lp_hbm.at[pl.ds(base * Ap, _RW * Ap)], lpv)
            lanes = lax.iota(jnp.int32, 16)
            nchunk = Ap // 16

            gd = lax.GatherDimensionNumbers(offset_dims=(),
                                            collapsed_slice_dims=(0,),
                                            start_index_map=(0,))

            def lane_take(x, idx):
                return lax.gather(
                    x, idx[:, None], gd, (1,),
                    mode=lax.GatherScatterMode.PROMISE_IN_BOUNDS)

            def bfly(x, op):
                for s in (1, 2, 4, 8):
                    x = op(x, lane_take(x, lanes ^ s))
                return x

            def row_body(r, carry):
                act_acc, alp_acc = carry

                def chunk_body(c, mi):
                    rmax, ridx, rlp = mi
                    v = yv[pl.ds(r * Ap + c * 16, 16)]
                    vlp = lpv[pl.ds(r * Ap + c * 16, 16)]
                    upd = v > rmax
                    rmax = jnp.where(upd, v, rmax)
                    ridx = jnp.where(upd, c * 16 + lanes, ridx)
                    rlp = jnp.where(upd, vlp, rlp)
                    return rmax, ridx, rlp

                rmax0 = jnp.full((16,), -3.4e38, jnp.float32)
                ridx0 = jnp.full((16,), 2 ** 30, jnp.int32)
                rlp0 = jnp.zeros((16,), jnp.float32)
                rmax, ridx, rlp = lax.fori_loop(0, nchunk, chunk_body,
                                                (rmax0, ridx0, rlp0))
                m = bfly(rmax, jnp.maximum)
                cand = jnp.where(rmax == m, ridx, 2 ** 30)
                gidx = bfly(cand, jnp.minimum)
                win = cand == gidx
                lpval = bfly(jnp.where(win, rlp, -3.4e38), jnp.maximum)
                act_acc = jnp.where(lanes == r, gidx, act_acc)
                alp_acc = jnp.where(lanes == r, lpval, alp_acc)
                return act_acc, alp_acc

            act0 = jnp.zeros((16,), jnp.int32)
            alp0 = jnp.zeros((16,), jnp.float32)
            act_acc, alp_acc = lax.fori_loop(0, _RW, row_body, (act0, alp0))
            actv[...] = act_acc
            alpv[...] = alp_acc
            pltpu.sync_copy(actv, act_hbm.at[pl.ds(base, _RW)])
            pltpu.sync_copy(alpv, alp_hbm.at[pl.ds(base, _RW)])

    return samp(y, lp)


def kernel(state, W1, b1, W2, b2, W3, b3, V1, vb1, V2, vb2, V3, vb3):
    B = state.shape[0]
    A = W3.shape[1]
    with jax.ensure_compile_time_eval():
        g = jax.random.gumbel(jax.random.key(42), (B, A), jnp.float32)

    probs, value, y, lp = _towers(state, W1, b1, W2, b2, W3, b3,
                                  V1, vb1, V2, vb2, V3, vb3, g)
    act, alp = _sc_sample(y.reshape(-1), lp.reshape(-1), B, _AP)
    return probs, value, act, alp


def kernel(state, W1, b1, W2, b2, W3, b3, V1, vb1, V2, vb2, V3, vb3):
    B = state.shape[0]
    A = W3.shape[1]
    with jax.ensure_compile_time_eval():
        g = jax.random.gumbel(jax.random.key(42), (B, A), jnp.float32)

    probs, value, y, lp = _towers(state, W1, b1, W2, b2, W3, b3,
                                  V1, vb1, V2, vb2, V3, vb3, g)
    act, alp = _sc_sample(y.reshape(-1), lp.reshape(-1), B, _AP)
    return probs, value, act, alp


# speedup vs baseline: 1.1880x; 1.1880x over previous
import jax
import jax.numpy as jnp
from jax.experimental import pallas as pl
from jax.experimental.pallas import tpu as pltpu

_BN = 256
_AB = 256


def _body(state_ref, w1_ref, b1_ref, w2_ref, b2_ref, w3_ref, b3_ref,
          v1_ref, vb1_ref, v2_ref, vb2_ref, v3_ref, vb3_ref, g_ref,
          probs_ref, value_ref, act_ref, alp_ref,
          sb, h1a, h2a, h1v, h2v, lg):
    B, S = state_ref.shape
    A = b3_ref.shape[1]
    H = h1a.shape[1]
    nb = H // _BN
    na = lg.shape[1] // _AB
    o2 = nb
    o3 = 2 * nb
    o4 = o3 + na

    i = pl.program_id(0)

    @pl.when(i == 0)
    def _cast_state():
        sb[...] = state_ref[...].astype(jnp.bfloat16)

    @pl.when(i < o2)
    def _p0():
        j = i
        xa = jnp.dot(sb[...], w1_ref[...], preferred_element_type=jnp.float32)
        h1a[:, pl.ds(j * _BN, _BN)] = jnp.tanh(xa + b1_ref[...]).astype(jnp.bfloat16)
        xv = jnp.dot(sb[...], v1_ref[...], preferred_element_type=jnp.float32)
        h1v[:, pl.ds(j * _BN, _BN)] = jnp.tanh(xv + vb1_ref[...]).astype(jnp.bfloat16)

    @pl.when((i >= o2) & (i < o3))
    def _p1():
        j = i - o2
        xa = jnp.dot(h1a[...], w2_ref[...], preferred_element_type=jnp.float32)
        h2a[:, pl.ds(j * _BN, _BN)] = jnp.tanh(xa + b2_ref[...]).astype(jnp.bfloat16)
        xv = jnp.dot(h1v[...], v2_ref[...], preferred_element_type=jnp.float32)
        h2v[:, pl.ds(j * _BN, _BN)] = jnp.tanh(xv + vb2_ref[...]).astype(jnp.bfloat16)

    @pl.when((i >= o3) & (i < o4))
    def _p2():
        j = i - o3
        lg[:, pl.ds(j * _AB, _AB)] = jnp.dot(
            h2a[...], w3_ref[...], preferred_element_type=jnp.float32)

    @pl.when(i == o4)
    def _fin():
        v3row = v3_ref[...].astype(jnp.bfloat16).astype(jnp.float32)
        hv = h2v[...].astype(jnp.float32)
        value_ref[...] = (jnp.sum(hv * v3row, axis=-1, keepdims=True)
                          + vb3_ref[...])
        logits = lg[:, :A] + b3_ref[...]
        m = jnp.max(logits, axis=-1, keepdims=True)
        e = jnp.exp(logits - m)
        p = e / jnp.sum(e, axis=-1, keepdims=True)
        probs_ref[...] = p
        lp = jnp.log(p + 1e-20)
        y = lp + g_ref[...]
        ym = jnp.max(y, axis=-1, keepdims=True)
        cols = jax.lax.broadcasted_iota(jnp.int32, (B, A), 1)
        idx = jnp.min(jnp.where(y == ym, cols, A), axis=-1, keepdims=True)
        act_ref[...] = idx
        alp_ref[...] = jnp.sum(jnp.where(cols == idx, lp, 0.0),
                               axis=-1, keepdims=True)


def kernel(state, W1, b1, W2, b2, W3, b3, V1, vb1, V2, vb2, V3, vb3):
    B, S = state.shape
    H = W1.shape[1]
    A = W3.shape[1]
    nb = H // _BN
    na = pl.cdiv(A, _AB)
    Ap = na * _AB
    o2, o3 = nb, 2 * nb
    o4 = o3 + na
    steps = o4 + 1

    with jax.ensure_compile_time_eval():
        g = jax.random.gumbel(jax.random.key(42), (B, A), jnp.float32)

    in_specs = [
        pl.BlockSpec((B, S), lambda i: (0, 0)),
        pl.BlockSpec((S, _BN), lambda i: (0, jnp.clip(i, 0, nb - 1))),
        pl.BlockSpec((1, _BN), lambda i: (0, jnp.clip(i, 0, nb - 1))),
        pl.BlockSpec((H, _BN), lambda i: (0, jnp.clip(i - o2, 0, nb - 1))),
        pl.BlockSpec((1, _BN), lambda i: (0, jnp.clip(i - o2, 0, nb - 1))),
        pl.BlockSpec((S, _AB), lambda i: (0, jnp.clip(i - o3, 0, na - 1))),
        pl.BlockSpec((1, A), lambda i: (0, 0)),
        pl.BlockSpec((S, _BN), lambda i: (0, jnp.clip(i, 0, nb - 1))),
        pl.BlockSpec((1, _BN), lambda i: (0, jnp.clip(i, 0, nb - 1))),
        pl.BlockSpec((H, _BN), lambda i: (0, jnp.clip(i - o2, 0, nb - 1))),
        pl.BlockSpec((1, _BN), lambda i: (0, jnp.clip(i - o2, 0, nb - 1))),
        pl.BlockSpec((1, S), lambda i: (0, 0)),
        pl.BlockSpec((1, 1), lambda i: (0, 0)),
        pl.BlockSpec((B, A), lambda i: (0, 0)),
    ]
    out_specs = [
        pl.BlockSpec((B, A), lambda i: (0, 0)),
        pl.BlockSpec((B, 1), lambda i: (0, 0)),
        pl.BlockSpec((B, 1), lambda i: (0, 0)),
        pl.BlockSpec((B, 1), lambda i: (0, 0)),
    ]
    out_shape = [
        jax.ShapeDtypeStruct((B, A), jnp.float32),
        jax.ShapeDtypeStruct((B, 1), jnp.float32),
        jax.ShapeDtypeStruct((B, 1), jnp.int32),
        jax.ShapeDtypeStruct((B, 1), jnp.float32),
    ]
    scratch_shapes = [
        pltpu.VMEM((B, S), jnp.bfloat16),
        pltpu.VMEM((B, H), jnp.bfloat16),
        pltpu.VMEM((B, H), jnp.bfloat16),
        pltpu.VMEM((B, H), jnp.bfloat16),
        pltpu.VMEM((B, H), jnp.bfloat16),
        pltpu.VMEM((B, Ap), jnp.float32),
    ]

    probs, value, act, alp = pl.pallas_call(
        _body,
        grid=(steps,),
        in_specs=in_specs,
        out_specs=out_specs,
        out_shape=out_shape,
        scratch_shapes=scratch_shapes,
    )(state, W1, b1.reshape(1, H), W2, b2.reshape(1, H),
      W3, b3.reshape(1, A), V1, vb1.reshape(1, H), V2, vb2.reshape(1, H),
      V3.reshape(1, S), vb3.reshape(1, 1), g)
    return probs, value, act[:, 0], alp[:, 0]
